# initial kernel scaffold (unmeasured)
import jax
import jax.numpy as jnp
from jax import lax
from jax.experimental import pallas as pl
from jax.experimental.pallas import tpu as pltpu

N_DEV = 32
B = 2
S = 256
D = 512
CH = S // N_DEV
H_LOC = 4
DH = 64
EPS = 1e-5

_CompilerParams = getattr(pltpu, "CompilerParams", None) or getattr(
    pltpu, "TPUCompilerParams"
)


def kernel(x, Wq, Wk, Wv, Wo, t_emb, W_mod, W_ff1, W_ff2):
    def body(
        x_ref, wq_ref, wk_ref, wv_ref, wo_ref, temb_ref, wmod_ref,
        wff1_ref, wff2_ref, out_ref,
        partial_ref, x1_ref, red_ref, rs_buf,
        rs_send, rs_recv, ag_send, ag_recv,
    ):
        my = lax.axis_index("i")

        barrier = pltpu.get_barrier_semaphore()
        for r in range(1, N_DEV):
            pl.semaphore_signal(
                barrier, inc=1,
                device_id=((my + r) % N_DEV,),
                device_id_type=pl.DeviceIdType.MESH,
            )
        pl.semaphore_wait(barrier, N_DEV - 1)

        mod = jnp.dot(temb_ref[:, :], wmod_ref[:, :],
                      preferred_element_type=jnp.float32)
        sa, sha, ga, sm_, shm, gm = [mod[:, i * D:(i + 1) * D] for i in range(6)]

        def ln_mod(h, scale, shift):
            mu = jnp.mean(h, axis=-1, keepdims=True)
            var = jnp.mean((h - mu) * (h - mu), axis=-1, keepdims=True)
            return (h - mu) * lax.rsqrt(var + EPS) * (1.0 + scale) + shift

        for b in range(B):
            xa = ln_mod(x_ref[b, :, :], sa[b:b + 1, :], sha[b:b + 1, :])
            q = jnp.dot(xa, wq_ref[:, :], preferred_element_type=jnp.float32)
            k = jnp.dot(xa, wk_ref[:, :], preferred_element_type=jnp.float32)
            v = jnp.dot(xa, wv_ref[:, :], preferred_element_type=jnp.float32)
            heads = []
            for h in range(H_LOC):
                qh = q[:, h * DH:(h + 1) * DH]
                kh = k[:, h * DH:(h + 1) * DH]
                vh = v[:, h * DH:(h + 1) * DH]
                s = lax.dot_general(
                    qh, kh, (((1,), (1,)), ((), ())),
                    preferred_element_type=jnp.float32,
                ) * 0.125
                mx = jnp.max(s, axis=-1, keepdims=True)
                p = jnp.exp(s - mx)
                lsum = jnp.sum(p, axis=-1, keepdims=True)
                heads.append(
                    jnp.dot(p, vh, preferred_element_type=jnp.float32) / lsum
                )
            attn = jnp.concatenate(heads, axis=-1)
            partial_ref[b, :, :] = jnp.dot(
                attn, wo_ref[:, :], preferred_element_type=jnp.float32
            )

        def all_reduce(dest_ref, combine):
            rs = []
            for r in range(1, N_DEV):
                tgt = (my + r) % N_DEV
                rd = pltpu.make_async_remote_copy(
                    src_ref=partial_ref.at[:, pl.ds(tgt * CH, CH), :],
                    dst_ref=rs_buf.at[r],
                    send_sem=rs_send.at[r],
                    recv_sem=rs_recv.at[r],
                    device_id=(tgt,),
                    device_id_type=pl.DeviceIdType.MESH,
                )
                rd.start()
                rs.append(rd)
            red = partial_ref[:, pl.ds(my * CH, CH), :]
            for r in range(1, N_DEV):
                rs[r - 1].wait_recv()
                red = red + rs_buf[r, :, :, :]
            final = combine(red)
            red_ref[:, :, :] = final
            dest_ref[:, pl.ds(my * CH, CH), :] = final
            for r in range(1, N_DEV):
                rs[r - 1].wait_send()
            ag = []
            for r in range(1, N_DEV):
                tgt = (my + r) % N_DEV
                rd = pltpu.make_async_remote_copy(
                    src_ref=red_ref,
                    dst_ref=dest_ref.at[:, pl.ds(my * CH, CH), :],
                    send_sem=ag_send.at[r],
                    recv_sem=ag_recv.at[r],
                    device_id=(tgt,),
                    device_id_type=pl.DeviceIdType.MESH,
                )
                rd.start()
                ag.append(rd)
            for r in range(1, N_DEV):
                ag[r - 1].wait_recv()
            for r in range(1, N_DEV):
                ag[r - 1].wait_send()

        def combine_attn(red):
            x0c = x_ref[:, pl.ds(my * CH, CH), :]
            return x0c + ga[:, None, :] * red

        all_reduce(x1_ref, combine_attn)

        for b in range(B):
            xm = ln_mod(x1_ref[b, :, :], sm_[b:b + 1, :], shm[b:b + 1, :])
            hh = jnp.dot(xm, wff1_ref[:, :], preferred_element_type=jnp.float32)
            hh = hh / (1.0 + jnp.exp(-hh))
            partial_ref[b, :, :] = jnp.dot(
                hh, wff2_ref[:, :], preferred_element_type=jnp.float32
            )

        def combine_ff(red):
            x1c = x1_ref[:, pl.ds(my * CH, CH), :]
            return x1c + gm[:, None, :] * red

        all_reduce(out_ref, combine_ff)

    return pl.pallas_call(
        body,
        out_shape=jax.ShapeDtypeStruct((B, S, D), jnp.float32),
        in_specs=[pl.BlockSpec(memory_space=pltpu.VMEM)] * 9,
        out_specs=pl.BlockSpec(memory_space=pltpu.VMEM),
        scratch_shapes=[
            pltpu.VMEM((B, S, D), jnp.float32),
            pltpu.VMEM((B, S, D), jnp.float32),
            pltpu.VMEM((B, CH, D), jnp.float32),
            pltpu.VMEM((N_DEV, B, CH, D), jnp.float32),
            pltpu.SemaphoreType.DMA((N_DEV,)),
            pltpu.SemaphoreType.DMA((N_DEV,)),
            pltpu.SemaphoreType.DMA((N_DEV,)),
            pltpu.SemaphoreType.DMA((N_DEV,)),
        ],
        compiler_params=_CompilerParams(collective_id=0),
    )(x, Wq, Wk, Wv, Wo, t_emb, W_mod, W_ff1, W_ff2)


# baseline (device time: 79162 ns/iter reference)
import jax
import jax.numpy as jnp
from jax import lax
from jax.experimental import pallas as pl
from jax.experimental.pallas import tpu as pltpu

N_DEV = 32
B = 2
S = 256
D = 512
CH = S // N_DEV
H_LOC = 4
DH = 64
EPS = 1e-5

_CompilerParams = getattr(pltpu, "CompilerParams", None) or getattr(
    pltpu, "TPUCompilerParams"
)

try:
    jax.block_until_ready(jax.jit(lambda a: a + 1.0)(jnp.zeros((8, 128), jnp.float32)))
except Exception:
    pass


def kernel(x, Wq, Wk, Wv, Wo, t_emb, W_mod, W_ff1, W_ff2):
    def body(
        x_ref, wq_ref, wk_ref, wv_ref, wo_ref, temb_ref, wmod_ref,
        wff1_ref, wff2_ref, out_ref,
        partial_ref, x1_ref, red_ref, rs_buf,
        rs_send, rs_recv, ag_send, ag_recv,
    ):
        my = lax.axis_index("i")

        barrier = pltpu.get_barrier_semaphore()
        for r in range(1, N_DEV):
            pl.semaphore_signal(
                barrier, inc=1,
                device_id=((my + r) % N_DEV,),
                device_id_type=pl.DeviceIdType.MESH,
            )
        pl.semaphore_wait(barrier, N_DEV - 1)

        mod = jnp.dot(temb_ref[:, :], wmod_ref[:, :],
                      preferred_element_type=jnp.float32)
        sa, sha, ga, sm_, shm, gm = [mod[:, i * D:(i + 1) * D] for i in range(6)]

        def ln_mod(h, scale, shift):
            mu = jnp.mean(h, axis=-1, keepdims=True)
            var = jnp.mean((h - mu) * (h - mu), axis=-1, keepdims=True)
            return (h - mu) * lax.rsqrt(var + EPS) * (1.0 + scale) + shift

        for b in range(B):
            xa = ln_mod(x_ref[b, :, :], sa[b:b + 1, :], sha[b:b + 1, :])
            q = jnp.dot(xa, wq_ref[:, :], preferred_element_type=jnp.float32)
            k = jnp.dot(xa, wk_ref[:, :], preferred_element_type=jnp.float32)
            v = jnp.dot(xa, wv_ref[:, :], preferred_element_type=jnp.float32)
            heads = []
            for h in range(H_LOC):
                qh = q[:, h * DH:(h + 1) * DH]
                kh = k[:, h * DH:(h + 1) * DH]
                vh = v[:, h * DH:(h + 1) * DH]
                s = lax.dot_general(
                    qh, kh, (((1,), (1,)), ((), ())),
                    preferred_element_type=jnp.float32,
                ) * 0.125
                mx = jnp.max(s, axis=-1, keepdims=True)
                p = jnp.exp(s - mx)
                lsum = jnp.sum(p, axis=-1, keepdims=True)
                heads.append(
                    jnp.dot(p, vh, preferred_element_type=jnp.float32) / lsum
                )
            attn = jnp.concatenate(heads, axis=-1)
            partial_ref[b, :, :] = jnp.dot(
                attn, wo_ref[:, :], preferred_element_type=jnp.float32
            )

        def all_reduce(dest_ref, combine):
            rs = []
            for r in range(1, N_DEV):
                tgt = (my + r) % N_DEV
                rd = pltpu.make_async_remote_copy(
                    src_ref=partial_ref.at[:, pl.ds(tgt * CH, CH), :],
                    dst_ref=rs_buf.at[r],
                    send_sem=rs_send.at[r],
                    recv_sem=rs_recv.at[r],
                    device_id=(tgt,),
                    device_id_type=pl.DeviceIdType.MESH,
                )
                rd.start()
                rs.append(rd)
            red = partial_ref[:, pl.ds(my * CH, CH), :]
            for r in range(1, N_DEV):
                rs[r - 1].wait_recv()
                red = red + rs_buf[r, :, :, :]
            final = combine(red)
            red_ref[:, :, :] = final
            dest_ref[:, pl.ds(my * CH, CH), :] = final
            for r in range(1, N_DEV):
                rs[r - 1].wait_send()
            ag = []
            for r in range(1, N_DEV):
                tgt = (my + r) % N_DEV
                rd = pltpu.make_async_remote_copy(
                    src_ref=red_ref,
                    dst_ref=dest_ref.at[:, pl.ds(my * CH, CH), :],
                    send_sem=ag_send.at[r],
                    recv_sem=ag_recv.at[r],
                    device_id=(tgt,),
                    device_id_type=pl.DeviceIdType.MESH,
                )
                rd.start()
                ag.append(rd)
            for r in range(1, N_DEV):
                ag[r - 1].wait_recv()
            for r in range(1, N_DEV):
                ag[r - 1].wait_send()

        def combine_attn(red):
            x0c = x_ref[:, pl.ds(my * CH, CH), :]
            return x0c + ga[:, None, :] * red

        all_reduce(x1_ref, combine_attn)

        for b in range(B):
            xm = ln_mod(x1_ref[b, :, :], sm_[b:b + 1, :], shm[b:b + 1, :])
            hh = jnp.dot(xm, wff1_ref[:, :], preferred_element_type=jnp.float32)
            hh = hh / (1.0 + jnp.exp(-hh))
            partial_ref[b, :, :] = jnp.dot(
                hh, wff2_ref[:, :], preferred_element_type=jnp.float32
            )

        def combine_ff(red):
            x1c = x1_ref[:, pl.ds(my * CH, CH), :]
            return x1c + gm[:, None, :] * red

        all_reduce(out_ref, combine_ff)

    return pl.pallas_call(
        body,
        out_shape=jax.ShapeDtypeStruct((B, S, D), jnp.float32),
        in_specs=[pl.BlockSpec(memory_space=pltpu.VMEM)] * 9,
        out_specs=pl.BlockSpec(memory_space=pltpu.VMEM),
        scratch_shapes=[
            pltpu.VMEM((B, S, D), jnp.float32),
            pltpu.VMEM((B, S, D), jnp.float32),
            pltpu.VMEM((B, CH, D), jnp.float32),
            pltpu.VMEM((N_DEV, B, CH, D), jnp.float32),
            pltpu.SemaphoreType.DMA((N_DEV,)),
            pltpu.SemaphoreType.DMA((N_DEV,)),
            pltpu.SemaphoreType.DMA((N_DEV,)),
            pltpu.SemaphoreType.DMA((N_DEV,)),
        ],
        compiler_params=_CompilerParams(collective_id=0),
    )(x, Wq, Wk, Wv, Wo, t_emb, W_mod, W_ff1, W_ff2)


# device time: 57486 ns/iter; 1.3771x vs baseline; 1.3771x over previous
import jax
import jax.numpy as jnp
from jax import lax
from jax.experimental import pallas as pl
from jax.experimental.pallas import tpu as pltpu

N_DEV = 32
B = 2
S = 256
D = 512
R = B * S
CHF = R // N_DEV
H_LOC = 4
DH = 64
EPS = 1e-5

_CompilerParams = getattr(pltpu, "CompilerParams", None) or getattr(
    pltpu, "TPUCompilerParams"
)

try:
    jax.block_until_ready(jax.jit(lambda a: a + 1.0)(jnp.zeros((8, 128), jnp.float32)))
except Exception:
    pass


def kernel(x, Wq, Wk, Wv, Wo, t_emb, W_mod, W_ff1, W_ff2):
    def body(
        x_ref, wq_ref, wk_ref, wv_ref, wo_ref, temb_ref, wmod_ref,
        wff1_ref, wff2_ref, out_ref,
        partial16_ref, x1_16_ref, out16_ref, red16_ref, rs_buf,
        rs_send, rs_recv, ag_send, ag_recv,
    ):
        my = lax.axis_index("i")

        barrier = pltpu.get_barrier_semaphore()
        for r in range(1, N_DEV):
            pl.semaphore_signal(
                barrier, inc=1,
                device_id=((my + r) % N_DEV,),
                device_id_type=pl.DeviceIdType.MESH,
            )
        pl.semaphore_wait(barrier, N_DEV - 1)

        mod = jnp.dot(temb_ref[:, :], wmod_ref[:, :],
                      preferred_element_type=jnp.float32)
        sa, sha, ga, sm_, shm, gm = [mod[:, i * D:(i + 1) * D] for i in range(6)]

        def ln_mod(h, scale, shift):
            mu = jnp.mean(h, axis=-1, keepdims=True)
            var = jnp.mean((h - mu) * (h - mu), axis=-1, keepdims=True)
            return (h - mu) * lax.rsqrt(var + EPS) * (1.0 + scale) + shift

        for b in range(B):
            xa = ln_mod(x_ref[b, :, :], sa[b:b + 1, :], sha[b:b + 1, :])
            q = jnp.dot(xa, wq_ref[:, :], preferred_element_type=jnp.float32)
            k = jnp.dot(xa, wk_ref[:, :], preferred_element_type=jnp.float32)
            v = jnp.dot(xa, wv_ref[:, :], preferred_element_type=jnp.float32)
            heads = []
            for h in range(H_LOC):
                qh = q[:, h * DH:(h + 1) * DH]
                kh = k[:, h * DH:(h + 1) * DH]
                vh = v[:, h * DH:(h + 1) * DH]
                s = lax.dot_general(
                    qh, kh, (((1,), (1,)), ((), ())),
                    preferred_element_type=jnp.float32,
                ) * 0.125
                mx = jnp.max(s, axis=-1, keepdims=True)
                p = jnp.exp(s - mx)
                lsum = jnp.sum(p, axis=-1, keepdims=True)
                heads.append(
                    jnp.dot(p, vh, preferred_element_type=jnp.float32) / lsum
                )
            attn = jnp.concatenate(heads, axis=-1)
            partial16_ref[pl.ds(b * S, S), :] = jnp.dot(
                attn, wo_ref[:, :], preferred_element_type=jnp.float32
            ).astype(jnp.bfloat16)

        b_idx = my // (S // CHF)
        s0 = (my % (S // CHF)) * CHF

        def pick_batch(v2):
            return jnp.where(b_idx == 0, v2[0:1, :], v2[1:2, :])

        def all_reduce(dest16_ref, combine):
            rs = []
            for r in range(1, N_DEV):
                tgt = (my + r) % N_DEV
                rd = pltpu.make_async_remote_copy(
                    src_ref=partial16_ref.at[pl.ds(tgt * CHF, CHF), :],
                    dst_ref=rs_buf.at[r],
                    send_sem=rs_send.at[r],
                    recv_sem=rs_recv.at[r],
                    device_id=(tgt,),
                    device_id_type=pl.DeviceIdType.MESH,
                )
                rd.start()
                rs.append(rd)
            red = partial16_ref[pl.ds(my * CHF, CHF), :].astype(jnp.float32)
            for r in range(1, N_DEV):
                rs[r - 1].wait_recv()
                red = red + rs_buf[r, :, :].astype(jnp.float32)
            final16 = combine(red).astype(jnp.bfloat16)
            red16_ref[:, :] = final16
            dest16_ref[pl.ds(my * CHF, CHF), :] = final16
            for r in range(1, N_DEV):
                rs[r - 1].wait_send()
            ag = []
            for r in range(1, N_DEV):
                tgt = (my + r) % N_DEV
                rd = pltpu.make_async_remote_copy(
                    src_ref=red16_ref,
                    dst_ref=dest16_ref.at[pl.ds(my * CHF, CHF), :],
                    send_sem=ag_send.at[r],
                    recv_sem=ag_recv.at[r],
                    device_id=(tgt,),
                    device_id_type=pl.DeviceIdType.MESH,
                )
                rd.start()
                ag.append(rd)
            for r in range(1, N_DEV):
                ag[r - 1].wait_recv()
            for r in range(1, N_DEV):
                ag[r - 1].wait_send()

        def combine_attn(red):
            x0c = jnp.where(
                b_idx == 0,
                x_ref[0, pl.ds(s0, CHF), :],
                x_ref[1, pl.ds(s0, CHF), :],
            )
            return x0c + pick_batch(ga) * red

        all_reduce(x1_16_ref, combine_attn)

        for b in range(B):
            x1b = x1_16_ref[pl.ds(b * S, S), :].astype(jnp.float32)
            xm = ln_mod(x1b, sm_[b:b + 1, :], shm[b:b + 1, :])
            hh = jnp.dot(xm, wff1_ref[:, :], preferred_element_type=jnp.float32)
            hh = hh / (1.0 + jnp.exp(-hh))
            partial16_ref[pl.ds(b * S, S), :] = jnp.dot(
                hh, wff2_ref[:, :], preferred_element_type=jnp.float32
            ).astype(jnp.bfloat16)

        def combine_ff(red):
            x1c = x1_16_ref[pl.ds(my * CHF, CHF), :].astype(jnp.float32)
            return x1c + pick_batch(gm) * red

        all_reduce(out16_ref, combine_ff)

        for b in range(B):
            out_ref[b, :, :] = out16_ref[pl.ds(b * S, S), :].astype(jnp.float32)

    return pl.pallas_call(
        body,
        out_shape=jax.ShapeDtypeStruct((B, S, D), jnp.float32),
        in_specs=[pl.BlockSpec(memory_space=pltpu.VMEM)] * 9,
        out_specs=pl.BlockSpec(memory_space=pltpu.VMEM),
        scratch_shapes=[
            pltpu.VMEM((R, D), jnp.bfloat16),
            pltpu.VMEM((R, D), jnp.bfloat16),
            pltpu.VMEM((R, D), jnp.bfloat16),
            pltpu.VMEM((CHF, D), jnp.bfloat16),
            pltpu.VMEM((N_DEV, CHF, D), jnp.bfloat16),
            pltpu.SemaphoreType.DMA((N_DEV,)),
            pltpu.SemaphoreType.DMA((N_DEV,)),
            pltpu.SemaphoreType.DMA((N_DEV,)),
            pltpu.SemaphoreType.DMA((N_DEV,)),
        ],
        compiler_params=_CompilerParams(collective_id=0),
    )(x, Wq, Wk, Wv, Wo, t_emb, W_mod, W_ff1, W_ff2)
